# Initial kernel scaffold; baseline (speedup 1.0000x reference)
import functools

import jax
import jax.numpy as jnp
from jax.experimental import pallas as pl
from jax.experimental.pallas import tpu as pltpu

S = 2048
D = 1024
E = 8
H = 2048

TB = 256
HB = 512


def _stage_a_body(x_ref, nw_ref, rw_ref, cos_ref, sin_ref, hid_ref, probs_ref):
    x = x_ref[...]
    nw = nw_ref[...]
    var = jnp.mean(x * x, axis=-1, keepdims=True)
    h = x * jax.lax.rsqrt(var + 1e-6) * nw
    half = D // 2
    x1 = h[:, :half]
    x2 = h[:, half:]
    cos = cos_ref[...]
    sin = sin_ref[...]
    rot1 = x1 * cos - x2 * sin
    rot2 = x2 * cos + x1 * sin
    h = jnp.concatenate([rot1, rot2], axis=-1)
    hid_ref[...] = h
    logits = jax.lax.dot_general(h, rw_ref[...], (((1,), (1,)), ((), ())),
                                 preferred_element_type=jnp.float32)
    m = jnp.max(logits, axis=-1, keepdims=True)
    p = jnp.exp(logits - m)
    p = p / jnp.sum(p, axis=-1, keepdims=True)
    lane = jax.lax.broadcasted_iota(jnp.int32, (TB, E), 1)
    i1 = jnp.argmax(p, axis=-1)
    m1 = lane == i1[:, None]
    v1 = jnp.max(p, axis=-1)
    neg = jnp.finfo(jnp.float32).min
    p_masked = jnp.where(m1, neg, p)
    i2 = jnp.argmax(p_masked, axis=-1)
    m2 = lane == i2[:, None]
    v2 = jnp.max(p_masked, axis=-1)
    denom = jnp.maximum(v1 + v2, 1e-8)
    probs_ref[...] = jnp.where(m1 | m2, p, 0.0) / denom[:, None]


def _stage_b_body(hid_ref, probs_ref, x_ref, w1_ref, w3_ref, w2_ref, out_ref):
    e = pl.program_id(0)
    hb = pl.program_id(1)

    @pl.when((e == 0) & (hb == 0))
    def _init():
        out_ref[...] = x_ref[...]

    h = hid_ref[...].astype(jnp.bfloat16)
    w1 = w1_ref[0].astype(jnp.bfloat16)
    w3 = w3_ref[0].astype(jnp.bfloat16)
    w2 = w2_ref[0].astype(jnp.bfloat16)
    h1 = jax.lax.dot_general(h, w1, (((1,), (1,)), ((), ())),
                             preferred_element_type=jnp.float32)
    h3 = jax.lax.dot_general(h, w3, (((1,), (1,)), ((), ())),
                             preferred_element_type=jnp.float32)
    g = (h1 * jax.lax.logistic(h1) * h3).astype(jnp.bfloat16)
    out_c = jax.lax.dot_general(g, w2, (((1,), (1,)), ((), ())),
                                preferred_element_type=jnp.float32)
    out_ref[...] += out_c * probs_ref[...]


@jax.jit
def _run(xs, norm_w, router_w, W1, W3, W2, cos, sin):
    hidden, probs = pl.pallas_call(
        _stage_a_body,
        grid=(S // TB,),
        in_specs=[
            pl.BlockSpec((TB, D), lambda t: (t, 0)),
            pl.BlockSpec((1, D), lambda t: (0, 0)),
            pl.BlockSpec((E, D), lambda t: (0, 0)),
            pl.BlockSpec((TB, D // 2), lambda t: (t, 0)),
            pl.BlockSpec((TB, D // 2), lambda t: (t, 0)),
        ],
        out_specs=[
            pl.BlockSpec((TB, D), lambda t: (t, 0)),
            pl.BlockSpec((TB, E), lambda t: (t, 0)),
        ],
        out_shape=[
            jax.ShapeDtypeStruct((S, D), jnp.float32),
            jax.ShapeDtypeStruct((S, E), jnp.float32),
        ],
    )(xs, norm_w.reshape(1, D), router_w, cos, sin)

    out = pl.pallas_call(
        _stage_b_body,
        grid=(E, H // HB),
        in_specs=[
            pl.BlockSpec((S, D), lambda e, hb: (0, 0)),
            pl.BlockSpec((S, 1), lambda e, hb: (0, e)),
            pl.BlockSpec((S, D), lambda e, hb: (0, 0)),
            pl.BlockSpec((1, HB, D), lambda e, hb: (e, hb, 0)),
            pl.BlockSpec((1, HB, D), lambda e, hb: (e, hb, 0)),
            pl.BlockSpec((1, D, HB), lambda e, hb: (e, 0, hb)),
        ],
        out_specs=pl.BlockSpec((S, D), lambda e, hb: (0, 0)),
        out_shape=jax.ShapeDtypeStruct((S, D), jnp.float32),
        compiler_params=pltpu.CompilerParams(
            dimension_semantics=("arbitrary", "arbitrary"),
        ),
    )(hidden, probs, xs, W1, W3, W2)
    return out


def kernel(x, norm_w, router_w, W1, W3, W2):
    B = x.shape[0]
    xs = x.reshape(S, D)
    half = D // 2
    inv_freq = 1.0 / (10000.0 ** (jnp.arange(0, half, dtype=jnp.float32) / half))
    pos = jnp.arange(S, dtype=jnp.float32)
    freqs = pos[:, None] * inv_freq[None, :]
    cos = jnp.cos(freqs)
    sin = jnp.sin(freqs)
    out = _run(xs, norm_w, router_w, W1, W3, W2, cos, sin)
    return out.reshape(B, S, D)


# speedup vs baseline: 1.0672x; 1.0672x over previous
import functools

import jax
import jax.numpy as jnp
from jax.experimental import pallas as pl
from jax.experimental.pallas import tpu as pltpu

S = 2048
D = 1024
E = 8
H = 2048

TB = 256
HB = 512


def _stage_a_body(x_ref, nw_ref, rw_ref, cos_ref, sin_ref, hid_ref, probs_ref):
    x = x_ref[...]
    nw = nw_ref[...]
    var = jnp.mean(x * x, axis=-1, keepdims=True)
    h = x * jax.lax.rsqrt(var + 1e-6) * nw
    half = D // 2
    x1 = h[:, :half]
    x2 = h[:, half:]
    cos = cos_ref[...]
    sin = sin_ref[...]
    rot1 = x1 * cos - x2 * sin
    rot2 = x2 * cos + x1 * sin
    h = jnp.concatenate([rot1, rot2], axis=-1)
    hid_ref[...] = h
    logits = jax.lax.dot_general(h, rw_ref[...], (((1,), (1,)), ((), ())),
                                 preferred_element_type=jnp.float32)
    m = jnp.max(logits, axis=-1, keepdims=True)
    p = jnp.exp(logits - m)
    p = p / jnp.sum(p, axis=-1, keepdims=True)
    lane = jax.lax.broadcasted_iota(jnp.int32, (TB, E), 1)
    i1 = jnp.argmax(p, axis=-1)
    m1 = lane == i1[:, None]
    v1 = jnp.max(p, axis=-1)
    neg = jnp.finfo(jnp.float32).min
    p_masked = jnp.where(m1, neg, p)
    i2 = jnp.argmax(p_masked, axis=-1)
    m2 = lane == i2[:, None]
    v2 = jnp.max(p_masked, axis=-1)
    denom = jnp.maximum(v1 + v2, 1e-8)
    probs_ref[...] = jnp.where(m1 | m2, p, 0.0) / denom[:, None]


def _stage_b_body(hid_ref, probs_ref, x_ref, w1_ref, w3_ref, w2_ref, out_ref):
    e = pl.program_id(0)
    hb = pl.program_id(1)

    @pl.when((e == 0) & (hb == 0))
    def _init():
        out_ref[...] = x_ref[...]

    h = hid_ref[...].astype(jnp.bfloat16)
    w1 = w1_ref[0].astype(jnp.bfloat16)
    w3 = w3_ref[0].astype(jnp.bfloat16)
    w2 = w2_ref[0].astype(jnp.bfloat16)
    h1 = jax.lax.dot_general(h, w1, (((1,), (1,)), ((), ())),
                             preferred_element_type=jnp.float32)
    h3 = jax.lax.dot_general(h, w3, (((1,), (1,)), ((), ())),
                             preferred_element_type=jnp.float32)
    g = (h1 * jax.lax.logistic(h1) * h3).astype(jnp.bfloat16)
    out_c = jax.lax.dot_general(g, w2, (((1,), (1,)), ((), ())),
                                preferred_element_type=jnp.float32)
    p = probs_ref[...]
    lane = jax.lax.broadcasted_iota(jnp.int32, (S, E), 1)
    col = jnp.sum(jnp.where(lane == e, p, 0.0), axis=1, keepdims=True)
    out_ref[...] += out_c * col


@jax.jit
def _run(xs, norm_w, router_w, W1, W3, W2, cos, sin):
    hidden, probs = pl.pallas_call(
        _stage_a_body,
        grid=(S // TB,),
        in_specs=[
            pl.BlockSpec((TB, D), lambda t: (t, 0)),
            pl.BlockSpec((1, D), lambda t: (0, 0)),
            pl.BlockSpec((E, D), lambda t: (0, 0)),
            pl.BlockSpec((TB, D // 2), lambda t: (t, 0)),
            pl.BlockSpec((TB, D // 2), lambda t: (t, 0)),
        ],
        out_specs=[
            pl.BlockSpec((TB, D), lambda t: (t, 0)),
            pl.BlockSpec((TB, E), lambda t: (t, 0)),
        ],
        out_shape=[
            jax.ShapeDtypeStruct((S, D), jnp.float32),
            jax.ShapeDtypeStruct((S, E), jnp.float32),
        ],
    )(xs, norm_w.reshape(1, D), router_w, cos, sin)

    out = pl.pallas_call(
        _stage_b_body,
        grid=(E, H // HB),
        in_specs=[
            pl.BlockSpec((S, D), lambda e, hb: (0, 0)),
            pl.BlockSpec((S, E), lambda e, hb: (0, 0)),
            pl.BlockSpec((S, D), lambda e, hb: (0, 0)),
            pl.BlockSpec((1, HB, D), lambda e, hb: (e, hb, 0)),
            pl.BlockSpec((1, HB, D), lambda e, hb: (e, hb, 0)),
            pl.BlockSpec((1, D, HB), lambda e, hb: (e, 0, hb)),
        ],
        out_specs=pl.BlockSpec((S, D), lambda e, hb: (0, 0)),
        out_shape=jax.ShapeDtypeStruct((S, D), jnp.float32),
        compiler_params=pltpu.CompilerParams(
            dimension_semantics=("arbitrary", "arbitrary"),
        ),
    )(hidden, probs, xs, W1, W3, W2)
    return out


def kernel(x, norm_w, router_w, W1, W3, W2):
    B = x.shape[0]
    xs = x.reshape(S, D)
    half = D // 2
    inv_freq = 1.0 / (10000.0 ** (jnp.arange(0, half, dtype=jnp.float32) / half))
    pos = jnp.arange(S, dtype=jnp.float32)
    freqs = pos[:, None] * inv_freq[None, :]
    cos = jnp.cos(freqs)
    sin = jnp.sin(freqs)
    out = _run(xs, norm_w, router_w, W1, W3, W2, cos, sin)
    return out.reshape(B, S, D)
